# baseline (speedup 1.0000x reference)
import functools

import jax
import jax.numpy as jnp
from jax import lax
from jax.experimental import pallas as pl
from jax.experimental.pallas import tpu as pltpu
from jax.experimental.pallas import tpu_sc as plsc

DIMS = 10000
LEVELS = 1000
POS = 784
BATCH = 8
CLASSES = 10

W = 80
NCHUNK = DIMS // W
NTILES = 32
PSTRIP = 56
NSTRIP = POS // PSTRIP
NV = W // 16

DTC = 9600
DBLK = 640
C0 = DTC // W
NSC = NCHUNK - C0
BPT = 2
TPC = BATCH // BPT


def _sc_encode_body(lev_hbm, pos_hbm, idx_hbm, out_hbm,
                    idxb, ilist, lrA, lrB, ptA, ptB, accb, semA, semB):
    wid = lax.axis_index("s") * 2 + lax.axis_index("c")
    ci = wid // TPC
    bg = wid % TPC
    c = C0 + ci
    b0 = bg * BPT

    @pl.when(wid < NSC * TPC)
    def _():
        pltpu.sync_copy(idx_hbm.at[pl.ds(b0, BPT)], idxb)

        for bl in range(BPT):
            for v in range(POS // 16):
                sl = pl.ds(v * 16, 16)
                ilist[bl, sl] = idxb[bl, sl] * NCHUNK + c

        zero = jnp.zeros((16,), jnp.float32)
        for bl in range(BPT):
            for j in range(NV):
                accb[bl, pl.ds(j * 16, 16)] = zero

        def copies(s, lr, pt, sem):
            cps = [
                pltpu.make_async_copy(
                    lev_hbm.at[ilist.at[bl, pl.ds(s * PSTRIP, PSTRIP)]],
                    lr.at[bl], sem)
                for bl in range(BPT)
            ]
            cps.append(pltpu.make_async_copy(
                pos_hbm.at[pl.ds(s * PSTRIP, PSTRIP), pl.ds(c * W, W)],
                pt, sem))
            return cps

        def issue(s, lr, pt, sem):
            for cp in copies(s, lr, pt, sem):
                cp.start()

        def drain(s, lr, pt, sem):
            for cp in copies(s, lr, pt, sem):
                cp.wait()

        def compute(lr, pt):
            def pbody(p, carry2):
                pv = [pt[p, pl.ds(j * 16, 16)] for j in range(NV)]
                for bl in range(BPT):
                    for j in range(NV):
                        lv = lr[bl, p, pl.ds(j * 16, 16)]
                        plsc.addupdate(accb.at[bl, pl.ds(j * 16, 16)],
                                       lv * pv[j])
                return carry2

            lax.fori_loop(0, PSTRIP, pbody, 0)

        issue(0, lrA, ptA, semA)

        def ring(g, carry1):
            s0 = 2 * g
            issue(s0 + 1, lrB, ptB, semB)
            drain(s0, lrA, ptA, semA)
            compute(lrA, ptA)

            @pl.when(s0 + 2 < NSTRIP)
            def _():
                issue(s0 + 2, lrA, ptA, semA)

            drain(s0 + 1, lrB, ptB, semB)
            compute(lrB, ptB)
            return carry1

        lax.fori_loop(0, NSTRIP // 2, ring, 0)
        pltpu.sync_copy(
            accb, out_hbm.at[pl.ds(b0, BPT), pl.ds((c - C0) * W, W)])


def _sc_encode(level_weight, position_weight, idx):
    lev_rows = level_weight.reshape(LEVELS * NCHUNK, W)
    mesh = plsc.VectorSubcoreMesh(core_axis_name="c", subcore_axis_name="s")
    sc_encode = pl.kernel(
        _sc_encode_body,
        out_type=jax.ShapeDtypeStruct((BATCH, DIMS - DTC), jnp.float32),
        mesh=mesh,
        compiler_params=pltpu.CompilerParams(use_tc_tiling_on_sc=False),
        scratch_types=[
            pltpu.VMEM((BPT, POS), jnp.int32),
            pltpu.VMEM((BPT, POS), jnp.int32),
            pltpu.VMEM((BPT, PSTRIP, W), jnp.float32),
            pltpu.VMEM((BPT, PSTRIP, W), jnp.float32),
            pltpu.VMEM((PSTRIP, W), jnp.float32),
            pltpu.VMEM((PSTRIP, W), jnp.float32),
            pltpu.VMEM((BPT, W), jnp.float32),
            pltpu.SemaphoreType.DMA,
            pltpu.SemaphoreType.DMA,
        ],
    )
    return sc_encode(lev_rows, position_weight, idx)


sc_encode_for_test = jax.jit(_sc_encode)


def _tc_encode_body(xt_ref, lev_ref, pos_ref, cw_ref, hv_sc_ref, out_ref):
    i = pl.program_id(0)
    lev_f8 = lev_ref[...].astype(jnp.float8_e4m3fn)
    pos_blk = pos_ref[...]
    lvl_iota = jax.lax.broadcasted_iota(jnp.int32, (POS, LEVELS), 1)
    encs = []
    for b in range(BATCH):
        xb = xt_ref[:, b : b + 1]
        idx = jnp.clip(jnp.round(xb * (LEVELS - 1)), 0, LEVELS - 1).astype(jnp.int32)
        onehot = (idx == lvl_iota).astype(jnp.float8_e4m3fn)
        g = jnp.dot(onehot, lev_f8, preferred_element_type=jnp.float32)
        s = jnp.sum(g * pos_blk, axis=0)[None, :]
        encs.append(jnp.where(s > 0, 1.0, -1.0).astype(jnp.float32))
    enc_blk = jnp.concatenate(encs, axis=0)
    cw_blk = cw_ref[:, pl.ds(i * DBLK, DBLK)]
    logit_blk = jax.lax.dot_general(
        enc_blk, cw_blk, (((1,), (1,)), ((), ())),
        preferred_element_type=jnp.float32)

    @pl.when(i == 0)
    def _():
        enc_sc = jnp.where(hv_sc_ref[...] > 0, 1.0, -1.0).astype(jnp.float32)
        out_ref[...] = logit_blk + jax.lax.dot_general(
            enc_sc, cw_ref[:, pl.ds(DTC, DIMS - DTC)],
            (((1,), (1,)), ((), ())), preferred_element_type=jnp.float32)

    @pl.when(i > 0)
    def _():
        out_ref[...] += logit_blk


def _finish_body(hv_tc_ref, hv_sc_ref, cw_ref, out_ref):
    cw = cw_ref[...]
    enc_tc = jnp.where(hv_tc_ref[...] > 0, 1.0, -1.0).astype(jnp.float32)
    enc_sc = jnp.where(hv_sc_ref[...] > 0, 1.0, -1.0).astype(jnp.float32)
    out_ref[...] = (
        jax.lax.dot_general(enc_tc, cw[:, :DTC], (((1,), (1,)), ((), ())),
                            preferred_element_type=jnp.float32)
        + jax.lax.dot_general(enc_sc, cw[:, DTC:], (((1,), (1,)), ((), ())),
                              preferred_element_type=jnp.float32))


@jax.jit
def kernel(x, position_weight, level_weight, classify_weight):
    xf = x.reshape(BATCH, POS)
    idx = jnp.clip(jnp.round(xf * (LEVELS - 1)), 0, LEVELS - 1).astype(jnp.int32)
    xt = xf.T

    hv_sc = _sc_encode(level_weight, position_weight, idx)

    logit = pl.pallas_call(
        _tc_encode_body,
        grid=((DTC + DBLK - 1) // DBLK,),
        in_specs=[
            pl.BlockSpec((POS, BATCH), lambda i: (0, 0)),
            pl.BlockSpec((LEVELS, DBLK), lambda i: (0, i)),
            pl.BlockSpec((POS, DBLK), lambda i: (0, i)),
            pl.BlockSpec((CLASSES, DIMS), lambda i: (0, 0)),
            pl.BlockSpec((BATCH, DIMS - DTC), lambda i: (0, 0)),
        ],
        out_specs=pl.BlockSpec((BATCH, CLASSES), lambda i: (0, 0)),
        out_shape=jax.ShapeDtypeStruct((BATCH, CLASSES), jnp.float32),
    )(xt, level_weight, position_weight, classify_weight, hv_sc)
    return logit


# speedup vs baseline: 1.1964x; 1.1964x over previous
import functools

import jax
import jax.numpy as jnp
from jax import lax
from jax.experimental import pallas as pl
from jax.experimental.pallas import tpu as pltpu
from jax.experimental.pallas import tpu_sc as plsc

DIMS = 10000
LEVELS = 1000
POS = 784
BATCH = 8
CLASSES = 10

W = 80
NCHUNK = DIMS // W
NTILES = 32
PSTRIP = 56
NSTRIP = POS // PSTRIP
NV = W // 16

DTC = 9600
DBLK = 640
C0 = DTC // W
NSC = NCHUNK - C0
BPT = 2
TPC = BATCH // BPT


def _sc_encode_body(lev_hbm, pos_hbm, idx_hbm, out_hbm,
                    idxb, ilist, lrA, lrB, ptA, ptB, accb, semA, semB):
    wid = lax.axis_index("s") * 2 + lax.axis_index("c")
    ci = wid // TPC
    bg = wid % TPC
    c = C0 + ci
    b0 = bg * BPT

    @pl.when(wid < NSC * TPC)
    def _():
        pltpu.sync_copy(idx_hbm.at[pl.ds(b0, BPT)], idxb)

        for bl in range(BPT):
            for v in range(POS // 16):
                sl = pl.ds(v * 16, 16)
                ilist[bl, sl] = idxb[bl, sl] * NCHUNK + c

        zero = jnp.zeros((16,), jnp.float32)
        for bl in range(BPT):
            for j in range(NV):
                accb[bl, pl.ds(j * 16, 16)] = zero

        def copies(s, lr, pt, sem):
            cps = [
                pltpu.make_async_copy(
                    lev_hbm.at[ilist.at[bl, pl.ds(s * PSTRIP, PSTRIP)]],
                    lr.at[bl], sem)
                for bl in range(BPT)
            ]
            cps.append(pltpu.make_async_copy(
                pos_hbm.at[pl.ds(s * PSTRIP, PSTRIP), pl.ds(c * W, W)],
                pt, sem))
            return cps

        def issue(s, lr, pt, sem):
            for cp in copies(s, lr, pt, sem):
                cp.start()

        def drain(s, lr, pt, sem):
            for cp in copies(s, lr, pt, sem):
                cp.wait()

        def compute(lr, pt):
            def pbody(p, carry2):
                pv = [pt[p, pl.ds(j * 16, 16)] for j in range(NV)]
                for bl in range(BPT):
                    for j in range(NV):
                        lv = lr[bl, p, pl.ds(j * 16, 16)]
                        plsc.addupdate(accb.at[bl, pl.ds(j * 16, 16)],
                                       lv * pv[j])
                return carry2

            lax.fori_loop(0, PSTRIP, pbody, 0)

        issue(0, lrA, ptA, semA)

        def ring(g, carry1):
            s0 = 2 * g
            issue(s0 + 1, lrB, ptB, semB)
            drain(s0, lrA, ptA, semA)
            compute(lrA, ptA)

            @pl.when(s0 + 2 < NSTRIP)
            def _():
                issue(s0 + 2, lrA, ptA, semA)

            drain(s0 + 1, lrB, ptB, semB)
            compute(lrB, ptB)
            return carry1

        lax.fori_loop(0, NSTRIP // 2, ring, 0)
        pltpu.sync_copy(
            accb, out_hbm.at[pl.ds(b0, BPT), pl.ds((c - C0) * W, W)])


def _sc_encode(level_weight, position_weight, idx):
    lev_rows = level_weight.reshape(LEVELS * NCHUNK, W)
    mesh = plsc.VectorSubcoreMesh(core_axis_name="c", subcore_axis_name="s")
    sc_encode = pl.kernel(
        _sc_encode_body,
        out_type=jax.ShapeDtypeStruct((BATCH, DIMS - DTC), jnp.float32),
        mesh=mesh,
        compiler_params=pltpu.CompilerParams(use_tc_tiling_on_sc=False),
        scratch_types=[
            pltpu.VMEM((BPT, POS), jnp.int32),
            pltpu.VMEM((BPT, POS), jnp.int32),
            pltpu.VMEM((BPT, PSTRIP, W), jnp.float32),
            pltpu.VMEM((BPT, PSTRIP, W), jnp.float32),
            pltpu.VMEM((PSTRIP, W), jnp.float32),
            pltpu.VMEM((PSTRIP, W), jnp.float32),
            pltpu.VMEM((BPT, W), jnp.float32),
            pltpu.SemaphoreType.DMA,
            pltpu.SemaphoreType.DMA,
        ],
    )
    return sc_encode(lev_rows, position_weight, idx)


sc_encode_for_test = jax.jit(_sc_encode)


def _tc_encode_body(xt_ref, lev_ref, pos_ref, out_ref):
    lev_f8 = lev_ref[...].astype(jnp.float8_e4m3fn)
    pos_blk = pos_ref[...]
    lvl_iota = jax.lax.broadcasted_iota(jnp.int32, (POS, LEVELS), 1)
    for b in range(BATCH):
        xb = xt_ref[:, b : b + 1]
        idx = jnp.clip(jnp.round(xb * (LEVELS - 1)), 0, LEVELS - 1).astype(jnp.int32)
        onehot = (idx == lvl_iota).astype(jnp.float8_e4m3fn)
        g = jnp.dot(onehot, lev_f8, preferred_element_type=jnp.float32)
        out_ref[b, :] = jnp.sum(g * pos_blk, axis=0)


def _finish_body(hv_tc_ref, hv_sc_ref, cw_ref, out_ref):
    cw = cw_ref[...]
    enc_tc = jnp.where(hv_tc_ref[...] > 0, 1.0, -1.0).astype(jnp.float32)
    enc_sc = jnp.where(hv_sc_ref[...] > 0, 1.0, -1.0).astype(jnp.float32)
    out_ref[...] = (
        jax.lax.dot_general(enc_tc, cw[:, :DTC], (((1,), (1,)), ((), ())),
                            preferred_element_type=jnp.float32)
        + jax.lax.dot_general(enc_sc, cw[:, DTC:], (((1,), (1,)), ((), ())),
                              preferred_element_type=jnp.float32))


@jax.jit
def kernel(x, position_weight, level_weight, classify_weight):
    xf = x.reshape(BATCH, POS)
    idx = jnp.clip(jnp.round(xf * (LEVELS - 1)), 0, LEVELS - 1).astype(jnp.int32)
    xt = xf.T

    hv_tc = pl.pallas_call(
        _tc_encode_body,
        grid=((DTC + DBLK - 1) // DBLK,),
        in_specs=[
            pl.BlockSpec((POS, BATCH), lambda i: (0, 0)),
            pl.BlockSpec((LEVELS, DBLK), lambda i: (0, i)),
            pl.BlockSpec((POS, DBLK), lambda i: (0, i)),
        ],
        out_specs=pl.BlockSpec((BATCH, DBLK), lambda i: (0, i)),
        out_shape=jax.ShapeDtypeStruct((BATCH, DTC), jnp.float32),
    )(xt, level_weight, position_weight)

    hv_sc = _sc_encode(level_weight, position_weight, idx)

    logit = pl.pallas_call(
        _finish_body,
        out_shape=jax.ShapeDtypeStruct((BATCH, CLASSES), jnp.float32),
    )(hv_tc, hv_sc, classify_weight)
    return logit
